# Initial kernel scaffold; baseline (speedup 1.0000x reference)
import jax
import jax.numpy as jnp
from jax.experimental import pallas as pl


def kernel(inp, edge_index, W1a, b1a, W2a, b2a, W1b, b1b, W2b, b2b):
    raise NotImplementedError("write your pallas kernel here")


# speedup vs baseline: 5.6720x; 5.6720x over previous
import functools

import jax
import jax.numpy as jnp
from jax import lax
from jax.experimental import pallas as pl
from jax.experimental.pallas import tpu as pltpu
from jax.experimental.pallas import tpu_sc as plsc

N = 10000
E = 320000
D = 128

NC = 2
NS = 16
NW = NC * NS
EPW = E // NW
C = 128
FULL = EPW // C
REM = EPW - FULL * C
RPT = 624
ECNT = E // NS
CFULL = ECNT // C
CREM = ECNT - CFULL * C

_f32 = jnp.float32


def _zeros16():
    return jnp.zeros((16,), _f32)


def _make_sc_segsum(with_count):
    out_type = [jax.ShapeDtypeStruct((NC, N, D), _f32)]
    if with_count:
        out_type.append(jax.ShapeDtypeStruct((N,), _f32))

    scratch_types = [
        pltpu.VMEM_SHARED((N, D), _f32),
        pltpu.VMEM_SHARED((N,), _f32),
        pltpu.VMEM((C,), jnp.int32),
        pltpu.VMEM((C,), jnp.int32),
        pltpu.VMEM((REM,), jnp.int32),
        pltpu.VMEM((REM,), jnp.int32),
        pltpu.VMEM((CREM,), jnp.int32),
        pltpu.VMEM((C, D), _f32),
        pltpu.VMEM((REM, D), _f32),
        pltpu.VMEM((C,), _f32),
        pltpu.VMEM((208, D), _f32),
        pltpu.VMEM((400,), _f32),
        pltpu.SemaphoreType.DMA,
    ]
    mesh = plsc.VectorSubcoreMesh(core_axis_name="c", subcore_axis_name="s")

    def body(y, esrc, edst, *rest):
        if with_count:
            out, cnt_out = rest[0], rest[1]
            rest = rest[2:]
        else:
            out = rest[0]
            rest = rest[1:]
        (acc, cntacc, sidx, didx, sidx16, didx16, didx32, rows, rows16,
         ones, zbuf, z1, sem) = rest
        c = lax.axis_index("c")
        s = lax.axis_index("s")

        def zrow(r, _):
            for j in range(8):
                zbuf[r, pl.ds(j * 16, 16)] = _zeros16()
            return 0
        lax.fori_loop(0, 208, zrow, 0)
        for k in range(3):
            pltpu.sync_copy(zbuf, acc.at[pl.ds(s * RPT + k * 208, 208)])

        @pl.when(s == 0)
        def _():
            pltpu.sync_copy(zbuf.at[pl.ds(0, 16)], acc.at[pl.ds(NS * RPT, 16)])

        if with_count:
            @pl.when(jnp.logical_and(c == 0, s == 0))
            def _():
                def z1row(i, _):
                    z1[pl.ds(i * 16, 16)] = _zeros16()
                    return 0
                lax.fori_loop(0, 25, z1row, 0)
                for k in range(25):
                    pltpu.sync_copy(z1, cntacc.at[pl.ds(k * 400, 400)])

            for j in range(8):
                ones[pl.ds(j * 16, 16)] = jnp.ones((16,), _f32)

        plsc.subcore_barrier()

        if with_count:
            @pl.when(c == 0)
            def _():
                cbase = s * ECNT

                def cchunk(k, _):
                    b = cbase + k * C
                    pltpu.sync_copy(edst.at[pl.ds(b, C)], didx)
                    pltpu.sync_copy(ones, cntacc.at[didx], add=True)
                    return 0
                lax.fori_loop(0, CFULL, cchunk, 0)
                b = cbase + CFULL * C
                pltpu.sync_copy(edst.at[pl.ds(b, CREM)], didx32)
                pltpu.sync_copy(ones.at[pl.ds(0, CREM)], cntacc.at[didx32],
                                add=True)

        wbase = (c * NS + s) * EPW

        def mchunk(k, _):
            b = wbase + k * C
            pltpu.sync_copy(esrc.at[pl.ds(b, C)], sidx)
            pltpu.async_copy(y.at[sidx], rows, sem).wait()
            pltpu.sync_copy(edst.at[pl.ds(b, C)], didx)
            pltpu.sync_copy(rows, acc.at[didx], add=True)
            return 0
        lax.fori_loop(0, FULL, mchunk, 0)
        b = wbase + FULL * C
        pltpu.sync_copy(esrc.at[pl.ds(b, REM)], sidx16)
        pltpu.async_copy(y.at[sidx16], rows16, sem).wait()
        pltpu.sync_copy(edst.at[pl.ds(b, REM)], didx16)
        pltpu.sync_copy(rows16, acc.at[didx16], add=True)

        plsc.subcore_barrier()

        pltpu.sync_copy(acc.at[pl.ds(s * RPT, RPT)],
                        out.at[c, pl.ds(s * RPT, RPT)])

        @pl.when(s == 0)
        def _():
            pltpu.sync_copy(acc.at[pl.ds(NS * RPT, 16)],
                            out.at[c, pl.ds(NS * RPT, 16)])
        if with_count:
            @pl.when(jnp.logical_and(c == 0, s == 0))
            def _():
                pltpu.sync_copy(cntacc, cnt_out)

    return pl.kernel(body, out_type=out_type, mesh=mesh,
                     scratch_types=scratch_types,
                     name="sc_segsum_cnt" if with_count else "sc_segsum")


_sc_segsum_cnt = _make_sc_segsum(True)
_sc_segsum = _make_sc_segsum(False)


BLK = 1000
GRID = N // BLK

_full = lambda shape: pl.BlockSpec(shape, lambda i: tuple(0 for _ in shape))
_rows = lambda: pl.BlockSpec((BLK, D), lambda i: (i, 0))


def _tc1_body(x_ref, w1_ref, b1_ref, w2_ref, h_ref, y_ref):
    x = x_ref[...]
    h_ref[...] = jnp.dot(x, w1_ref[...], preferred_element_type=_f32) + b1_ref[...]
    y_ref[...] = jnp.dot(x, w2_ref[...], preferred_element_type=_f32)


_tc1 = pl.pallas_call(
    _tc1_body,
    grid=(GRID,),
    in_specs=[_rows(), _full((D, D)), _full((1, D)), _full((D, D))],
    out_specs=[_rows(), _rows()],
    out_shape=[jax.ShapeDtypeStruct((N, D), _f32)] * 2,
    compiler_params=pltpu.CompilerParams(dimension_semantics=("parallel",)),
)


def _combine(sp, cnt, b2):
    s = sp[0] + sp[1]
    return (s + cnt * b2) / jnp.maximum(cnt, 1.0)


def _tc2_body(h1_ref, sp_ref, cnt_ref, b2a_ref, w1_ref, b1_ref, w2_ref,
              h_ref, y_ref):
    mean = _combine(sp_ref[...], cnt_ref[...], b2a_ref[...])
    x2 = jnp.maximum(h1_ref[...] + mean, 0.0)
    h_ref[...] = jnp.dot(x2, w1_ref[...], preferred_element_type=_f32) + b1_ref[...]
    y_ref[...] = jnp.dot(x2, w2_ref[...], preferred_element_type=_f32)


_tc2 = pl.pallas_call(
    _tc2_body,
    grid=(GRID,),
    in_specs=[_rows(),
              pl.BlockSpec((2, BLK, D), lambda i: (0, i, 0)),
              pl.BlockSpec((BLK, 1), lambda i: (i, 0)),
              _full((1, D)), _full((D, D)), _full((1, D)), _full((D, D))],
    out_specs=[_rows(), _rows()],
    out_shape=[jax.ShapeDtypeStruct((N, D), _f32)] * 2,
    compiler_params=pltpu.CompilerParams(dimension_semantics=("parallel",)),
)


def _tc3_body(h2_ref, sp_ref, cnt_ref, b2b_ref, o_ref):
    mean = _combine(sp_ref[...], cnt_ref[...], b2b_ref[...])
    o_ref[...] = h2_ref[...] + mean


_tc3 = pl.pallas_call(
    _tc3_body,
    grid=(GRID,),
    in_specs=[_rows(),
              pl.BlockSpec((2, BLK, D), lambda i: (0, i, 0)),
              pl.BlockSpec((BLK, 1), lambda i: (i, 0)),
              _full((1, D))],
    out_specs=_rows(),
    out_shape=jax.ShapeDtypeStruct((N, D), _f32),
    compiler_params=pltpu.CompilerParams(dimension_semantics=("parallel",)),
)


def kernel(inp, edge_index, W1a, b1a, W2a, b2a, W1b, b1b, W2b, b2b):
    ei = edge_index.astype(jnp.int32)
    esrc, edst = ei[0], ei[1]
    h1, y1 = _tc1(inp, W1a, b1a.reshape(1, D), W2a)
    sp1, cnt = _sc_segsum_cnt(y1, esrc, edst)
    cnt2 = cnt.reshape(N, 1)
    h2, y2 = _tc2(h1, sp1, cnt2, b2a.reshape(1, D), W1b, b1b.reshape(1, D),
                  W2b)
    (sp2,) = _sc_segsum(y2, esrc, edst)
    return _tc3(h2, sp2, cnt2, b2b.reshape(1, D))
